# baseline (speedup 1.0000x reference)
import functools

import jax
import jax.numpy as jnp
from jax import lax
from jax.experimental import pallas as pl
from jax.experimental.pallas import tpu as pltpu
from jax.experimental.pallas import tpu_sc as plsc

D = 32
VOC4 = 250000
BATCH = 4096
SEQ = 200
NC = 2
NS = 16
NW = NC * NS
BT = BATCH // NW
ST = SEQ // 8
DT = D // 8
NB = 4
JB = BT // 16
DB = D // 16
NBLK = JB * DB

_mesh = plsc.VectorSubcoreMesh(core_axis_name="c", subcore_axis_name="s")


@functools.partial(
    pl.kernel,
    out_type=(
        jax.ShapeDtypeStruct((SEQ, DT, NW, 8 * BT), jnp.float32),
        jax.ShapeDtypeStruct((DT, NW, 8 * BT), jnp.float32),
    ),
    mesh=_mesh,
    compiler_params=pltpu.CompilerParams(
        use_tc_tiling_on_sc=False, needs_layout_passes=False),
    scratch_types=[
        pltpu.VMEM((ST, 8, BT), jnp.int32),
        pltpu.VMEM((NB, 2, 64), jnp.int32),
        pltpu.VMEM((NB, 2, 64, 128), jnp.float32),
        pltpu.VMEM((NB, DT, 8 * BT), jnp.float32),
        pltpu.VMEM((NBLK * 16, 16), jnp.float32),
        pltpu.VMEM((DT, 8 * BT), jnp.float32),
        pltpu.SemaphoreType.DMA,
        pltpu.SemaphoreType.DMA,
        pltpu.SemaphoreType.DMA,
        pltpu.SemaphoreType.DMA,
        pltpu.SemaphoreType.DMA,
        pltpu.SemaphoreType.DMA,
        pltpu.SemaphoreType.DMA,
        pltpu.SemaphoreType.DMA,
    ],
)
def _embed_pool(ids_hbm, table2_hbm, x_hbm, mean_hbm,
                idx_v, sidx_v, buf_v, tbuf_v, macc_v, mtb_v,
                g0, g1, g2, g3, w0, w1, w2, w3):
    gsems = (g0, g1, g2, g3)
    wsems = (w0, w1, w2, w3)
    wid = lax.axis_index("s") * NC + lax.axis_index("c")
    inv = jnp.float32(1.0 / SEQ)
    zero16 = jnp.zeros((16,), jnp.float32)
    lane = lax.iota(jnp.int32, 16)
    j0lanes = [jb * 16 + lane for jb in range(JB)]

    pltpu.sync_copy(ids_hbm.at[:, wid], idx_v)

    def zstep(r, _):
        macc_v[r, :] = zero16
        return 0
    lax.fori_loop(0, NBLK * 16, zstep, 0)

    def start_gather(s, b):
        for h in range(2):
            for q in range(4):
                sidx_v[b, h, pl.ds(q * 16, 16)] = (
                    idx_v[s // 8, s % 8, pl.ds(h * 64 + q * 16, 16)] >> 2)
            pltpu.async_copy(
                table2_hbm.at[sidx_v.at[b, h]],
                buf_v.at[b, h], gsems[b])

    for b in range(NB):
        start_gather(b, b)

    @pl.loop(0, SEQ, step=NB)
    def _round(s0):
        for b in range(NB):
            s = s0 + b
            tbuf = tbuf_v.at[b]
            for h in range(2):
                pltpu.make_async_copy(
                    table2_hbm.at[pl.ds(0, 64)], buf_v.at[b, h],
                    gsems[b]).wait()

            @pl.when(s >= NB)
            def _():
                pltpu.make_async_copy(
                    x_hbm.at[0, :, 0], tbuf, wsems[b]).wait()

            pcols = []
            for jb in range(JB):
                pv = idx_v[s // 8, s % 8, pl.ds(jb * 16, 16)]
                pcols.append((pv & 3) << 5)

            def tstep(t, _, b=b, tbuf=tbuf, pcols=pcols):
                rot = (lane + t) & 15
                rot3 = rot >> 3
                rotcol = (rot & 7) << 7
                for db in range(DB):
                    dvec = db * 16 + rot
                    dtv = db * 2 + rot3
                    for jb in range(JB):
                        bufh = buf_v.at[b, jb // 4]
                        j0l = (jb % 4) * 16
                        v = plsc.load_gather(
                            bufh, [j0l + lane, pcols[jb] + dvec])
                        plsc.store_scatter(
                            tbuf, [dtv, rotcol + j0lanes[jb]], v)
                        plsc.addupdate(
                            macc_v.at[(jb * DB + db) * 16 + t, :], v)
                return 0
            lax.fori_loop(0, 16, tstep, 0)

            pltpu.async_copy(tbuf, x_hbm.at[s, :, wid], wsems[b])

            @pl.when(s + NB < SEQ)
            def _():
                start_gather(s + NB, b)

    for b in range(NB):
        pltpu.make_async_copy(x_hbm.at[0, :, 0], tbuf_v.at[b], wsems[b]).wait()

    def mstep(t, _):
        rot = (lane + t) & 15
        rot3 = rot >> 3
        rotcol = (rot & 7) << 7
        for db in range(DB):
            dtv = db * 2 + rot3
            for jb in range(JB):
                v = macc_v[(jb * DB + db) * 16 + t, :] * inv
                plsc.store_scatter(mtb_v, [dtv, rotcol + j0lanes[jb]], v)
        return 0
    lax.fori_loop(0, 16, mstep, 0)
    pltpu.sync_copy(mtb_v, mean_hbm.at[:, wid])


def kernel(input_ids, embedding_weight):
    ids5 = input_ids.T.reshape(ST, 8, NW, BT).transpose(0, 2, 1, 3)
    wt2 = lax.optimization_barrier(embedding_weight.reshape(VOC4, 128))
    x5, m4 = _embed_pool(ids5, wt2)
    x = (x5.reshape(SEQ, DT, NW, 8, BT)
         .transpose(2, 4, 0, 1, 3).reshape(BATCH, SEQ, D))
    mean = (m4.reshape(DT, NW, 8, BT)
            .transpose(1, 3, 0, 2).reshape(BATCH, D))
    return x, mean


# speedup vs baseline: 1.0051x; 1.0051x over previous
import functools

import jax
import jax.numpy as jnp
from jax import lax
from jax.experimental import pallas as pl
from jax.experimental.pallas import tpu as pltpu
from jax.experimental.pallas import tpu_sc as plsc

D = 32
BATCH = 4096
SEQ = 200
NC = 2
NS = 16
NW = NC * NS
BT = BATCH // NW
ST = SEQ // 8
DT = D // 8
NB = 8
JB = BT // 16
DB = D // 16
NBLK = JB * DB

_mesh = plsc.VectorSubcoreMesh(core_axis_name="c", subcore_axis_name="s")


@functools.partial(
    pl.kernel,
    out_type=(
        jax.ShapeDtypeStruct((SEQ, DT, NW, 8 * BT), jnp.float32),
        jax.ShapeDtypeStruct((DT, NW, 8 * BT), jnp.float32),
    ),
    mesh=_mesh,
    compiler_params=pltpu.CompilerParams(
        use_tc_tiling_on_sc=False, needs_layout_passes=False),
    scratch_types=[
        pltpu.VMEM((ST, 8, BT), jnp.int32),
        pltpu.VMEM((NB, BT, D), jnp.float32),
        pltpu.VMEM((NB, DT, 8 * BT), jnp.float32),
        pltpu.VMEM((NBLK * 16, 16), jnp.float32),
        pltpu.VMEM((DT, 8 * BT), jnp.float32),
        pltpu.SemaphoreType.DMA,
        pltpu.SemaphoreType.DMA,
        pltpu.SemaphoreType.DMA,
        pltpu.SemaphoreType.DMA,
        pltpu.SemaphoreType.DMA,
        pltpu.SemaphoreType.DMA,
        pltpu.SemaphoreType.DMA,
        pltpu.SemaphoreType.DMA,
        pltpu.SemaphoreType.DMA,
        pltpu.SemaphoreType.DMA,
        pltpu.SemaphoreType.DMA,
        pltpu.SemaphoreType.DMA,
        pltpu.SemaphoreType.DMA,
        pltpu.SemaphoreType.DMA,
        pltpu.SemaphoreType.DMA,
        pltpu.SemaphoreType.DMA,
    ],
)
def _embed_pool(ids_hbm, table_hbm, x_hbm, mean_hbm,
                idx_v, buf_v, tbuf_v, macc_v, mtb_v,
                g0, g1, g2, g3, g4, g5, g6, g7, w0, w1, w2, w3, w4, w5, w6, w7):
    gsems = (g0, g1, g2, g3, g4, g5, g6, g7)
    wsems = (w0, w1, w2, w3, w4, w5, w6, w7)
    wid = lax.axis_index("s") * NC + lax.axis_index("c")
    inv = jnp.float32(1.0 / SEQ)
    zero16 = jnp.zeros((16,), jnp.float32)
    lane = lax.iota(jnp.int32, 16)
    j0lanes = [jb * 16 + lane for jb in range(JB)]

    pltpu.sync_copy(ids_hbm.at[:, wid], idx_v)

    def zstep(r, _):
        macc_v[r, :] = zero16
        return 0
    lax.fori_loop(0, NBLK * 16, zstep, 0)

    def start_gather(s, b):
        pltpu.async_copy(table_hbm.at[idx_v.at[s // 8, s % 8]],
                         buf_v.at[b], gsems[b])

    for b in range(NB):
        start_gather(b, b)

    @pl.loop(0, SEQ, step=NB)
    def _round(s0):
        for b in range(NB):
            s = s0 + b
            buf = buf_v.at[b]
            tbuf = tbuf_v.at[b]
            pltpu.make_async_copy(
                table_hbm.at[pl.ds(0, BT)], buf, gsems[b]).wait()

            @pl.when(s >= NB)
            def _():
                pltpu.make_async_copy(
                    x_hbm.at[0, :, 0], tbuf, wsems[b]).wait()

            def tstep(t, _, buf=buf, tbuf=tbuf):
                rot = (lane + t) & 15
                rot3 = rot >> 3
                rotcol = (rot & 7) << 7
                for db in range(DB):
                    dvec = db * 16 + rot
                    dtv = db * 2 + rot3
                    for jb in range(JB):
                        v = plsc.load_gather(buf, [j0lanes[jb], dvec])
                        plsc.store_scatter(
                            tbuf, [dtv, rotcol + j0lanes[jb]], v)
                        plsc.addupdate(
                            macc_v.at[(jb * DB + db) * 16 + t, :], v)
                return 0
            lax.fori_loop(0, 16, tstep, 0)

            pltpu.async_copy(tbuf, x_hbm.at[s, :, wid], wsems[b])

            @pl.when(s + NB < SEQ)
            def _():
                start_gather(s + NB, b)

    for b in range(NB):
        pltpu.make_async_copy(x_hbm.at[0, :, 0], tbuf_v.at[b], wsems[b]).wait()

    def mstep(t, _):
        rot = (lane + t) & 15
        rot3 = rot >> 3
        rotcol = (rot & 7) << 7
        for db in range(DB):
            dtv = db * 2 + rot3
            for jb in range(JB):
                v = macc_v[(jb * DB + db) * 16 + t, :] * inv
                plsc.store_scatter(mtb_v, [dtv, rotcol + j0lanes[jb]], v)
        return 0
    lax.fori_loop(0, 16, mstep, 0)
    pltpu.sync_copy(mtb_v, mean_hbm.at[:, wid])


def kernel(input_ids, embedding_weight):
    ids5 = input_ids.T.reshape(ST, 8, NW, BT).transpose(0, 2, 1, 3)
    x5, m4 = _embed_pool(ids5, embedding_weight)
    x = (x5.reshape(SEQ, DT, NW, 8, BT)
         .transpose(2, 4, 0, 1, 3).reshape(BATCH, SEQ, D))
    mean = (m4.reshape(DT, NW, 8, BT)
            .transpose(1, 3, 0, 2).reshape(BATCH, D))
    return x, mean


# speedup vs baseline: 1.0142x; 1.0090x over previous
import functools

import jax
import jax.numpy as jnp
from jax import lax
from jax.experimental import pallas as pl
from jax.experimental.pallas import tpu as pltpu
from jax.experimental.pallas import tpu_sc as plsc

D = 32
BATCH = 4096
SEQ = 200
NC = 2
NS = 16
NW = NC * NS
BT = BATCH // NW
ST = SEQ // 8
DT = D // 8
NB = 5
JB = BT // 16
DB = D // 16
NBLK = JB * DB

_mesh = plsc.VectorSubcoreMesh(core_axis_name="c", subcore_axis_name="s")


@functools.partial(
    pl.kernel,
    out_type=(
        jax.ShapeDtypeStruct((SEQ, DT, NW, 8 * BT), jnp.float32),
        jax.ShapeDtypeStruct((DT, NW, 8 * BT), jnp.float32),
    ),
    mesh=_mesh,
    compiler_params=pltpu.CompilerParams(
        use_tc_tiling_on_sc=False, needs_layout_passes=False),
    scratch_types=[
        pltpu.VMEM((ST, 8, BT), jnp.int32),
        pltpu.VMEM((NB, BT, D), jnp.float32),
        pltpu.VMEM((NB, DT, 8 * BT), jnp.float32),
        pltpu.VMEM((NBLK * 16, 16), jnp.float32),
        pltpu.VMEM((DT, 8 * BT), jnp.float32),
        pltpu.SemaphoreType.DMA,
        pltpu.SemaphoreType.DMA,
        pltpu.SemaphoreType.DMA,
        pltpu.SemaphoreType.DMA,
        pltpu.SemaphoreType.DMA,
        pltpu.SemaphoreType.DMA,
        pltpu.SemaphoreType.DMA,
        pltpu.SemaphoreType.DMA,
        pltpu.SemaphoreType.DMA,
        pltpu.SemaphoreType.DMA,
    ],
)
def _embed_pool(ids_hbm, table_hbm, x_hbm, mean_hbm,
                idx_v, buf_v, tbuf_v, macc_v, mtb_v,
                g0, g1, g2, g3, g4, w0, w1, w2, w3, w4):
    gsems = (g0, g1, g2, g3, g4)
    wsems = (w0, w1, w2, w3, w4)
    wid = lax.axis_index("s") * NC + lax.axis_index("c")
    inv = jnp.float32(1.0 / SEQ)
    zero16 = jnp.zeros((16,), jnp.float32)
    lane = lax.iota(jnp.int32, 16)
    j0lanes = [jb * 16 + lane for jb in range(JB)]

    pltpu.sync_copy(ids_hbm.at[:, wid], idx_v)

    def zstep(r, _):
        macc_v[r, :] = zero16
        return 0
    lax.fori_loop(0, NBLK * 16, zstep, 0)

    def start_gather(s, b):
        pltpu.async_copy(table_hbm.at[idx_v.at[s // 8, s % 8]],
                         buf_v.at[b], gsems[b])

    for b in range(NB):
        start_gather(b, b)

    @pl.loop(0, SEQ, step=NB)
    def _round(s0):
        for b in range(NB):
            s = s0 + b
            buf = buf_v.at[b]
            tbuf = tbuf_v.at[b]
            pltpu.make_async_copy(
                table_hbm.at[pl.ds(0, BT)], buf, gsems[b]).wait()

            @pl.when(s >= NB)
            def _():
                pltpu.make_async_copy(
                    x_hbm.at[0, :, 0], tbuf, wsems[b]).wait()

            def tstep(t, _, buf=buf, tbuf=tbuf):
                rot = (lane + t) & 15
                rot3 = rot >> 3
                rotcol = (rot & 7) << 7
                for db in range(DB):
                    dvec = db * 16 + rot
                    dtv = db * 2 + rot3
                    for jb in range(JB):
                        v = plsc.load_gather(buf, [j0lanes[jb], dvec])
                        plsc.store_scatter(
                            tbuf, [dtv, rotcol + j0lanes[jb]], v)
                        plsc.addupdate(
                            macc_v.at[(jb * DB + db) * 16 + t, :], v)
                return 0
            lax.fori_loop(0, 16, tstep, 0)

            pltpu.async_copy(tbuf, x_hbm.at[s, :, wid], wsems[b])

            @pl.when(s + NB < SEQ)
            def _():
                start_gather(s + NB, b)

    for b in range(NB):
        pltpu.make_async_copy(x_hbm.at[0, :, 0], tbuf_v.at[b], wsems[b]).wait()

    def mstep(t, _):
        rot = (lane + t) & 15
        rot3 = rot >> 3
        rotcol = (rot & 7) << 7
        for db in range(DB):
            dtv = db * 2 + rot3
            for jb in range(JB):
                v = macc_v[(jb * DB + db) * 16 + t, :] * inv
                plsc.store_scatter(mtb_v, [dtv, rotcol + j0lanes[jb]], v)
        return 0
    lax.fori_loop(0, 16, mstep, 0)
    pltpu.sync_copy(mtb_v, mean_hbm.at[:, wid])


def kernel(input_ids, embedding_weight):
    ids5 = input_ids.T.reshape(ST, 8, NW, BT).transpose(0, 2, 1, 3)
    x5, m4 = _embed_pool(ids5, embedding_weight)
    x = (x5.reshape(SEQ, DT, NW, 8, BT)
         .transpose(2, 4, 0, 1, 3).reshape(BATCH, SEQ, D))
    mean = (m4.reshape(DT, NW, 8, BT)
            .transpose(1, 3, 0, 2).reshape(BATCH, D))
    return x, mean
